# baseline (speedup 1.0000x reference)
import functools

import jax
import jax.numpy as jnp
from jax import lax
from jax.experimental import pallas as pl
from jax.experimental.pallas import tpu as pltpu
from jax.experimental.pallas import tpu_sc as plsc

_H = 4
_K = 5
_TAU = 0.25
_NEG = -1e30

_SC_CHUNK = 320


def _sc_allmax(x, tmp, off):
    for k in (8, 4, 2, 1):
        tmp[pl.ds(off, 16)] = x
        tmp[pl.ds(off + 16, 16)] = x
        x = jnp.maximum(x, tmp[pl.ds(off + k, 16)])
    return x


def _sc_row_thresholds(vs, tmp, off):
    def vmax16(xs):
        while len(xs) > 1:
            xs = [jnp.maximum(xs[2 * i], xs[2 * i + 1]) for i in range(len(xs) // 2)]
        return xs[0]

    neg = jnp.full((16,), _NEG, jnp.float32)
    t1b = _sc_allmax(vmax16(vs), tmp, off)
    tb = t1b
    for _ in range(_K - 1):
        tb = _sc_allmax(vmax16([jnp.where(v < tb, v, neg) for v in vs]), tmp, off)
    return t1b, tb


def _sc_thresholds(p_flat, n):
    mesh = plsc.VectorSubcoreMesh(core_axis_name="c", subcore_axis_name="s")
    sub = 80

    @functools.partial(
        pl.kernel,
        mesh=mesh,
        out_type=jax.ShapeDtypeStruct((n * 16,), jnp.float32),
        scratch_types=[
            pltpu.VMEM((_H * sub * 256,), jnp.float32),
            pltpu.VMEM((sub * 16,), jnp.float32),
            pltpu.VMEM((_H * 32,), jnp.float32),
        ],
    )
    def thr_kernel(p_hbm, thr_hbm, buf, stage, tmp):
        wid = lax.axis_index("s") * 2 + lax.axis_index("c")
        n0 = jnp.minimum(wid * _SC_CHUNK, n - _SC_CHUNK)
        lane = lax.iota(jnp.int32, 16)
        for c in range(_SC_CHUNK // sub):
            nc = n0 + c * sub
            for h in range(_H):
                pltpu.sync_copy(
                    p_hbm.at[pl.ds((h * n + nc) * 256, sub * 256)],
                    buf.at[pl.ds(h * sub * 256, sub * 256)],
                )

            def body(r, carry):
                vec = jnp.zeros((16,), jnp.float32)
                for h in range(_H):
                    vs = [
                        buf[pl.ds((h * sub + r) * 256 + 16 * k, 16)]
                        for k in range(16)
                    ]
                    t1b, t5b = _sc_row_thresholds(vs, tmp, 32 * h)
                    vec = jnp.where(lane == jnp.full((16,), h, jnp.int32), t1b, vec)
                    vec = jnp.where(lane == jnp.full((16,), 4 + h, jnp.int32), t5b, vec)
                stage[pl.ds(r * 16, 16)] = vec
                return carry

            lax.fori_loop(0, sub, body, 0)
            pltpu.sync_copy(stage, thr_hbm.at[pl.ds(nc * 16, sub * 16)])

    return thr_kernel(p_flat).reshape(n, 16)


def _top5_sel(p):
    t1 = jnp.max(p, axis=-1, keepdims=True)
    t = t1
    for _ in range(_K - 1):
        t = jnp.max(jnp.where(p < t, p, _NEG), axis=-1, keepdims=True)
    e = jnp.exp((p - t1) * (1.0 / _TAU))
    u = jnp.where(p >= t, e, 0.0)
    denom = jnp.sum(u, axis=-1, keepdims=True)
    return u / denom


def _dot(x, y):
    return jax.lax.dot_general(
        x, y, (((1,), (0,)), ((), ())), preferred_element_type=jnp.float32
    )


def _scatter_accum(j, hw, p1_ref, acc_ref, ws_ref):
    @pl.when(j == 0)
    def _():
        acc_ref[...] = jnp.zeros_like(acc_ref)
        ws_ref[...] = jnp.zeros_like(ws_ref)

    dh = hw.shape[1] // _H
    hn, b, a = p1_ref.shape
    s_all = _top5_sel(p1_ref[...].reshape(hn * b, a))
    for h in range(_H):
        s = s_all[h * b:(h + 1) * b, :]
        hwh = hw[:, h * dh:(h + 1) * dh]
        acc_ref[h] = acc_ref[h] + _dot(s.T, hwh)
        ws_ref[h:h + 1, :] = ws_ref[h:h + 1, :] + jnp.sum(s, axis=0, keepdims=True)


def _gather_heads(p2_ref, acc_ref, ws_ref):
    hn, b, a = p2_ref.shape
    s_all = _top5_sel(p2_ref[...].reshape(hn * b, a))
    outs = []
    for h in range(_H):
        wsum = jnp.maximum(ws_ref[h], 1e-6).reshape(-1, 1)
        anchor = acc_ref[h] / wsum
        outs.append(_dot(s_all[h * b:(h + 1) * b, :], anchor))
    return jnp.concatenate(outs, axis=1)


def _k_first(x_ref, w_ref, p1_ref, acc_ref, ws_ref):
    hw = _dot(x_ref[...], w_ref[...])
    _scatter_accum(pl.program_id(0), hw, p1_ref, acc_ref, ws_ref)


def _k_mid(p2_ref, acc_in_ref, ws_in_ref, b_ref, w_ref, p1_ref, acc_ref, ws_ref):
    h = _gather_heads(p2_ref, acc_in_ref, ws_in_ref) + b_ref[...]
    h = jnp.where(h > 0, h, jnp.exp(jnp.minimum(h, 0.0)) - 1.0)
    hw = _dot(h, w_ref[...])
    _scatter_accum(pl.program_id(0), hw, p1_ref, acc_ref, ws_ref)


def _k_last(p2_ref, thr_ref, acc_in_ref, ws_in_ref, b_ref, y_ref):
    outs = []
    for h in range(_H):
        wsum = jnp.maximum(ws_in_ref[h], 1e-6).reshape(-1, 1)
        anchor = acc_in_ref[h] / wsum
        p = p2_ref[h]
        t1 = thr_ref[:, h:h + 1]
        t5 = thr_ref[:, 4 + h:5 + h]
        e = jnp.exp((p - t1) * (1.0 / _TAU))
        u = jnp.where(p >= t5, e, 0.0)
        s = u / jnp.sum(u, axis=-1, keepdims=True)
        outs.append(_dot(s, anchor))
    h = jnp.concatenate(outs, axis=1) + b_ref[...]
    m = jnp.max(h, axis=-1, keepdims=True)
    z = h - m
    y_ref[...] = z - jnp.log(jnp.sum(jnp.exp(z), axis=-1, keepdims=True))


def _anchor_spec(dh):
    return pl.BlockSpec((_H, 256, dh), lambda j: (0, 0, 0))


def _ws_spec():
    return pl.BlockSpec((8, 256), lambda j: (0, 0))


def _params():
    return pltpu.CompilerParams(dimension_semantics=("arbitrary",))


def kernel(x, edge_index, W0, b0, P1_0, P2_0, W1, b1, P1_1, P2_1, W2, b2, P1_2, P2_2):
    del edge_index
    n, din = x.shape
    a = P1_0.shape[2]
    bsz = 1000 if n % 1000 == 0 else n
    nb = n // bsz
    f32 = jnp.float32

    hid = W0.shape[1]
    out = W2.shape[1]
    dh0 = hid // _H
    dh1 = W1.shape[1] // _H
    dh2 = out // _H

    pblk = lambda: pl.BlockSpec((_H, bsz, a), lambda j: (0, j, 0))

    thr2 = _sc_thresholds(P2_2.reshape(_H * n * a), n)

    acc0, ws0 = pl.pallas_call(
        _k_first,
        grid=(nb,),
        in_specs=[
            pl.BlockSpec((bsz, din), lambda j: (j, 0)),
            pl.BlockSpec((din, hid), lambda j: (0, 0)),
            pblk(),
        ],
        out_specs=[_anchor_spec(dh0), _ws_spec()],
        out_shape=[
            jax.ShapeDtypeStruct((_H, a, dh0), f32),
            jax.ShapeDtypeStruct((8, a), f32),
        ],
        compiler_params=_params(),
    )(x, W0, P1_0)

    def mid(p2, acc_in, ws_in, bvec, w, p1, dh_out):
        return pl.pallas_call(
            _k_mid,
            grid=(nb,),
            in_specs=[
                pblk(),
                _anchor_spec(acc_in.shape[2]),
                _ws_spec(),
                pl.BlockSpec((1, bvec.shape[0]), lambda j: (0, 0)),
                pl.BlockSpec(w.shape, lambda j: (0, 0)),
                pblk(),
            ],
            out_specs=[_anchor_spec(dh_out), _ws_spec()],
            out_shape=[
                jax.ShapeDtypeStruct((_H, a, dh_out), f32),
                jax.ShapeDtypeStruct((8, a), f32),
            ],
            compiler_params=_params(),
        )(p2, acc_in, ws_in, bvec.reshape(1, -1), w, p1)

    acc1, ws1 = mid(P2_0, acc0, ws0, b0, W1, P1_1, dh1)
    acc2, ws2 = mid(P2_1, acc1, ws1, b1, W2, P1_2, dh2)

    y = pl.pallas_call(
        _k_last,
        grid=(nb,),
        in_specs=[
            pblk(),
            pl.BlockSpec((bsz, 16), lambda j: (j, 0)),
            _anchor_spec(dh2),
            _ws_spec(),
            pl.BlockSpec((1, out), lambda j: (0, 0)),
        ],
        out_specs=pl.BlockSpec((bsz, out), lambda j: (j, 0)),
        out_shape=jax.ShapeDtypeStruct((n, out), f32),
        compiler_params=_params(),
    )(P2_2, thr2, acc2, ws2, b2.reshape(1, -1))

    return y


# speedup vs baseline: 1.1063x; 1.1063x over previous
import functools

import jax
import jax.numpy as jnp
from jax import lax
from jax.experimental import pallas as pl
from jax.experimental.pallas import tpu as pltpu
from jax.experimental.pallas import tpu_sc as plsc

_H = 4
_K = 5
_TAU = 0.25
_NEG = -1e30

_SC_CHUNK = 320


def _sc_allmax(x, tmp, off):
    for k in (8, 4, 2, 1):
        tmp[pl.ds(off, 16)] = x
        tmp[pl.ds(off + 16, 16)] = x
        x = jnp.maximum(x, tmp[pl.ds(off + k, 16)])
    return x


def _sc_row_thresholds(vs, tmp, off):
    def vmax16(xs):
        while len(xs) > 1:
            xs = [jnp.maximum(xs[2 * i], xs[2 * i + 1]) for i in range(len(xs) // 2)]
        return xs[0]

    neg = jnp.full((16,), _NEG, jnp.float32)
    t1b = _sc_allmax(vmax16(vs), tmp, off)
    tb = t1b
    for _ in range(_K - 1):
        tb = _sc_allmax(vmax16([jnp.where(v < tb, v, neg) for v in vs]), tmp, off)
    return t1b, tb


def _sc_thresholds(p_flat, n):
    mesh = plsc.VectorSubcoreMesh(core_axis_name="c", subcore_axis_name="s")
    sub = 80

    @functools.partial(
        pl.kernel,
        mesh=mesh,
        out_type=jax.ShapeDtypeStruct((n * 16,), jnp.float32),
        scratch_types=[
            pltpu.VMEM((_H * sub, 256), jnp.float32),
            pltpu.VMEM((sub * 16,), jnp.float32),
            pltpu.VMEM((_H * 32,), jnp.float32),
        ],
    )
    def thr_kernel(p_hbm, thr_hbm, buf, stage, tmp):
        wid = lax.axis_index("s") * 2 + lax.axis_index("c")
        n0 = jnp.minimum(wid * _SC_CHUNK, n - _SC_CHUNK)
        lane = lax.iota(jnp.int32, 16)
        for c in range(_SC_CHUNK // sub):
            nc = n0 + c * sub
            for h in range(_H):
                pltpu.sync_copy(
                    p_hbm.at[h, pl.ds(nc, sub), :],
                    buf.at[pl.ds(h * sub, sub), :],
                )

            def body(r, carry):
                vec = jnp.zeros((16,), jnp.float32)
                for h in range(_H):
                    vs = [
                        buf[h * sub + r, pl.ds(16 * k, 16)]
                        for k in range(16)
                    ]
                    t1b, t5b = _sc_row_thresholds(vs, tmp, 32 * h)
                    vec = jnp.where(lane == jnp.full((16,), h, jnp.int32), t1b, vec)
                    vec = jnp.where(lane == jnp.full((16,), 4 + h, jnp.int32), t5b, vec)
                stage[pl.ds(r * 16, 16)] = vec
                return carry

            lax.fori_loop(0, sub, body, 0)
            pltpu.sync_copy(stage, thr_hbm.at[pl.ds(nc * 16, sub * 16)])

    return thr_kernel(p_flat).reshape(n, 16)


def _top5_sel(p):
    t1 = jnp.max(p, axis=-1, keepdims=True)
    t = t1
    for _ in range(_K - 1):
        t = jnp.max(jnp.where(p < t, p, _NEG), axis=-1, keepdims=True)
    e = jnp.exp((p - t1) * (1.0 / _TAU))
    u = jnp.where(p >= t, e, 0.0)
    denom = jnp.sum(u, axis=-1, keepdims=True)
    return u / denom


def _dot(x, y):
    return jax.lax.dot_general(
        x, y, (((1,), (0,)), ((), ())), preferred_element_type=jnp.float32
    )


def _scatter_accum(j, hw, p1_ref, acc_ref, ws_ref):
    @pl.when(j == 0)
    def _():
        acc_ref[...] = jnp.zeros_like(acc_ref)
        ws_ref[...] = jnp.zeros_like(ws_ref)

    dh = hw.shape[1] // _H
    hn, b, a = p1_ref.shape
    s_all = _top5_sel(p1_ref[...].reshape(hn * b, a))
    for h in range(_H):
        s = s_all[h * b:(h + 1) * b, :]
        hwh = hw[:, h * dh:(h + 1) * dh]
        acc_ref[h] = acc_ref[h] + _dot(s.T, hwh)
        ws_ref[h:h + 1, :] = ws_ref[h:h + 1, :] + jnp.sum(s, axis=0, keepdims=True)


def _gather_heads(p2_ref, acc_ref, ws_ref):
    hn, b, a = p2_ref.shape
    s_all = _top5_sel(p2_ref[...].reshape(hn * b, a))
    outs = []
    for h in range(_H):
        wsum = jnp.maximum(ws_ref[h], 1e-6).reshape(-1, 1)
        anchor = acc_ref[h] / wsum
        outs.append(_dot(s_all[h * b:(h + 1) * b, :], anchor))
    return jnp.concatenate(outs, axis=1)


def _k_first(x_ref, w_ref, p1_ref, acc_ref, ws_ref):
    hw = _dot(x_ref[...], w_ref[...])
    _scatter_accum(pl.program_id(0), hw, p1_ref, acc_ref, ws_ref)


def _k_mid(p2_ref, acc_in_ref, ws_in_ref, b_ref, w_ref, p1_ref, acc_ref, ws_ref):
    h = _gather_heads(p2_ref, acc_in_ref, ws_in_ref) + b_ref[...]
    h = jnp.where(h > 0, h, jnp.exp(jnp.minimum(h, 0.0)) - 1.0)
    hw = _dot(h, w_ref[...])
    _scatter_accum(pl.program_id(0), hw, p1_ref, acc_ref, ws_ref)


def _k_last(p2_ref, thr_ref, acc_in_ref, ws_in_ref, b_ref, y_ref):
    outs = []
    for h in range(_H):
        wsum = jnp.maximum(ws_in_ref[h], 1e-6).reshape(-1, 1)
        anchor = acc_in_ref[h] / wsum
        p = p2_ref[h]
        t1 = thr_ref[:, h:h + 1]
        t5 = thr_ref[:, 4 + h:5 + h]
        e = jnp.exp((p - t1) * (1.0 / _TAU))
        u = jnp.where(p >= t5, e, 0.0)
        s = u / jnp.sum(u, axis=-1, keepdims=True)
        outs.append(_dot(s, anchor))
    h = jnp.concatenate(outs, axis=1) + b_ref[...]
    m = jnp.max(h, axis=-1, keepdims=True)
    z = h - m
    y_ref[...] = z - jnp.log(jnp.sum(jnp.exp(z), axis=-1, keepdims=True))


def _anchor_spec(dh):
    return pl.BlockSpec((_H, 256, dh), lambda j: (0, 0, 0))


def _ws_spec():
    return pl.BlockSpec((8, 256), lambda j: (0, 0))


def _params():
    return pltpu.CompilerParams(dimension_semantics=("arbitrary",))


def kernel(x, edge_index, W0, b0, P1_0, P2_0, W1, b1, P1_1, P2_1, W2, b2, P1_2, P2_2):
    del edge_index
    n, din = x.shape
    a = P1_0.shape[2]
    bsz = 1000 if n % 1000 == 0 else n
    nb = n // bsz
    f32 = jnp.float32

    hid = W0.shape[1]
    out = W2.shape[1]
    dh0 = hid // _H
    dh1 = W1.shape[1] // _H
    dh2 = out // _H

    pblk = lambda: pl.BlockSpec((_H, bsz, a), lambda j: (0, j, 0))

    thr2 = _sc_thresholds(P2_2, n)

    acc0, ws0 = pl.pallas_call(
        _k_first,
        grid=(nb,),
        in_specs=[
            pl.BlockSpec((bsz, din), lambda j: (j, 0)),
            pl.BlockSpec((din, hid), lambda j: (0, 0)),
            pblk(),
        ],
        out_specs=[_anchor_spec(dh0), _ws_spec()],
        out_shape=[
            jax.ShapeDtypeStruct((_H, a, dh0), f32),
            jax.ShapeDtypeStruct((8, a), f32),
        ],
        compiler_params=_params(),
    )(x, W0, P1_0)

    def mid(p2, acc_in, ws_in, bvec, w, p1, dh_out):
        return pl.pallas_call(
            _k_mid,
            grid=(nb,),
            in_specs=[
                pblk(),
                _anchor_spec(acc_in.shape[2]),
                _ws_spec(),
                pl.BlockSpec((1, bvec.shape[0]), lambda j: (0, 0)),
                pl.BlockSpec(w.shape, lambda j: (0, 0)),
                pblk(),
            ],
            out_specs=[_anchor_spec(dh_out), _ws_spec()],
            out_shape=[
                jax.ShapeDtypeStruct((_H, a, dh_out), f32),
                jax.ShapeDtypeStruct((8, a), f32),
            ],
            compiler_params=_params(),
        )(p2, acc_in, ws_in, bvec.reshape(1, -1), w, p1)

    acc1, ws1 = mid(P2_0, acc0, ws0, b0, W1, P1_1, dh1)
    acc2, ws2 = mid(P2_1, acc1, ws1, b1, W2, P1_2, dh2)

    y = pl.pallas_call(
        _k_last,
        grid=(nb,),
        in_specs=[
            pblk(),
            pl.BlockSpec((bsz, 16), lambda j: (j, 0)),
            _anchor_spec(dh2),
            _ws_spec(),
            pl.BlockSpec((1, out), lambda j: (0, 0)),
        ],
        out_specs=pl.BlockSpec((bsz, out), lambda j: (j, 0)),
        out_shape=jax.ShapeDtypeStruct((n, out), f32),
        compiler_params=_params(),
    )(P2_2, thr2, acc2, ws2, b2.reshape(1, -1))

    return y
